# baseline (speedup 1.0000x reference)
import jax
import jax.numpy as jnp
from jax import lax
from jax.experimental import pallas as pl
from jax.experimental.pallas import tpu as pltpu
from jax.experimental.pallas import tpu_sc as plsc

B = 16384
NC, NS = 2, 16
NW = NC * NS
BPW = B // NW
CHUNK = 2048
NCHUNK = B // CHUNK
EPS = 1e-5

_TABLE_DIMS = (32, 16, 16, 32)


def _sc_gather_body(t_user, t_country, t_device, t_interest,
                    i_user, i_country, i_device, i_interest,
                    o_user, o_country, o_device, o_interest,
                    iv0, iv1, iv2, iv3, rv0, rv1, rv2, rv3, sem):
    wid = lax.axis_index("s") * NC + lax.axis_index("c")
    base = wid * BPW
    tables = (t_user, t_country, t_device, t_interest)
    idxs = (i_user, i_country, i_device, i_interest)
    outs = (o_user, o_country, o_device, o_interest)
    ivs = (iv0, iv1, iv2, iv3)
    rvs = (rv0, rv1, rv2, rv3)
    for ih, iv in zip(idxs, ivs):
        pltpu.sync_copy(ih.at[pl.ds(base, BPW)], iv)
    copies = [pltpu.async_copy(t.at[iv], rv, sem)
              for t, iv, rv in zip(tables, ivs, rvs)]
    for c in copies:
        c.wait()
    for rv, o in zip(rvs, outs):
        pltpu.sync_copy(rv, o.at[pl.ds(base, BPW)])


def _sc_gather(tables, idxs):
    mesh = plsc.VectorSubcoreMesh(core_axis_name="c", subcore_axis_name="s")
    out_type = [jax.ShapeDtypeStruct((B, d), jnp.float32) for d in _TABLE_DIMS]
    scratch = ([pltpu.VMEM((BPW,), jnp.int32) for _ in _TABLE_DIMS]
               + [pltpu.VMEM((BPW, d), jnp.float32) for d in _TABLE_DIMS]
               + [pltpu.SemaphoreType.DMA])
    fn = pl.kernel(_sc_gather_body, out_type=out_type, mesh=mesh,
                   scratch_types=scratch,
                   compiler_params=pltpu.CompilerParams(
                       use_tc_tiling_on_sc=False))
    return fn(*tables, *idxs)


def _mlp_body(eu, ec, ed, ei, nm,
              W1r, b1r, g1r, bt1r, W2r, b2r, g2r, bt2r, W3r, b3r,
              out_ref, h1s, h2s, s1, q1, s2, q2):
    f32 = jnp.float32
    p = pl.program_id(0)
    c = pl.program_id(1)
    sl = pl.ds(c * CHUNK, CHUNK)

    @pl.when(p == 0)
    def _phase0():
        h = (jnp.dot(eu[...], W1r[0:32, :], preferred_element_type=f32)
             + jnp.dot(ec[...], W1r[32:48, :], preferred_element_type=f32)
             + jnp.dot(ed[...], W1r[48:64, :], preferred_element_type=f32)
             + jnp.dot(ei[...], W1r[64:96, :], preferred_element_type=f32)
             + jnp.dot(nm[...], W1r[96:112, :], preferred_element_type=f32)
             + b1r[...])
        h = jnp.maximum(h, 0.0)
        h1s[sl, :] = h
        hs = jnp.sum(h, axis=0, keepdims=True)
        hq = jnp.sum(h * h, axis=0, keepdims=True)

        @pl.when(c == 0)
        def _():
            s1[...] = hs
            q1[...] = hq

        @pl.when(c != 0)
        def _():
            s1[...] += hs
            q1[...] += hq

    @pl.when(p == 1)
    def _phase1():
        m1 = s1[...] * (1.0 / B)
        v1 = jnp.maximum(q1[...] * (1.0 / B) - m1 * m1, 0.0)
        a1 = lax.rsqrt(v1 + EPS) * g1r[...]
        c1 = bt1r[...] - m1 * a1
        hn = h1s[sl, :] * a1 + c1
        h2 = jnp.maximum(jnp.dot(hn, W2r[...], preferred_element_type=f32)
                         + b2r[...], 0.0)
        h2s[sl, :] = h2
        hs = jnp.sum(h2, axis=0, keepdims=True)
        hq = jnp.sum(h2 * h2, axis=0, keepdims=True)

        @pl.when(c == 0)
        def _():
            s2[...] = hs
            q2[...] = hq

        @pl.when(c != 0)
        def _():
            s2[...] += hs
            q2[...] += hq

    @pl.when(p == 2)
    def _phase2():
        m2 = s2[...] * (1.0 / B)
        v2 = jnp.maximum(q2[...] * (1.0 / B) - m2 * m2, 0.0)
        a2 = lax.rsqrt(v2 + EPS) * g2r[...]
        c2 = bt2r[...] - m2 * a2
        hn = h2s[sl, :] * a2 + c2
        out_ref[...] = (jnp.dot(hn, W3r[...], preferred_element_type=f32)
                        + b3r[...])


def _mlp_tc(eu, ec, ed, ei, nm, W1, b1, g1, bt1, W2, b2, g2, bt2, W3, b3):
    def chunk_spec(d):
        return pl.BlockSpec((CHUNK, d),
                            lambda p, c: (jnp.where(p == 0, c, 0), 0))

    def full_spec(shape):
        return pl.BlockSpec(shape, lambda p, c: (0, 0))

    return pl.pallas_call(
        _mlp_body,
        grid=(3, NCHUNK),
        in_specs=[
            chunk_spec(32), chunk_spec(16), chunk_spec(16), chunk_spec(32),
            chunk_spec(16),
            full_spec((112, 256)), full_spec((1, 256)), full_spec((1, 256)),
            full_spec((1, 256)),
            full_spec((256, 128)), full_spec((1, 128)), full_spec((1, 128)),
            full_spec((1, 128)),
            full_spec((128, 64)), full_spec((1, 64)),
        ],
        out_specs=pl.BlockSpec((CHUNK, 64),
                               lambda p, c: (jnp.where(p == 2, c, 0), 0)),
        out_shape=jax.ShapeDtypeStruct((B, 64), jnp.float32),
        scratch_shapes=[
            pltpu.VMEM((B, 256), jnp.float32),
            pltpu.VMEM((B, 128), jnp.float32),
            pltpu.VMEM((1, 256), jnp.float32),
            pltpu.VMEM((1, 256), jnp.float32),
            pltpu.VMEM((1, 128), jnp.float32),
            pltpu.VMEM((1, 128), jnp.float32),
        ],
    )(eu, ec, ed, ei, nm, W1, b1, g1, bt1, W2, b2, g2, bt2, W3, b3)


def kernel(cat_user_id, cat_country, cat_device, cat_interest,
           numerical_inputs,
           T_user, T_country, T_device, T_interest,
           W1, b1, g1, bt1, W2, b2, g2, bt2, W3, b3):
    idxs = (cat_user_id[:, 0], cat_country[:, 0], cat_device[:, 0],
            cat_interest[:, 0])
    eu, ec, ed, ei = _sc_gather((T_user, T_country, T_device, T_interest),
                                idxs)
    return _mlp_tc(eu, ec, ed, ei, numerical_inputs,
                   W1, b1.reshape(1, 256), g1.reshape(1, 256),
                   bt1.reshape(1, 256), W2, b2.reshape(1, 128),
                   g2.reshape(1, 128), bt2.reshape(1, 128),
                   W3, b3.reshape(1, 64))
